# Initial kernel scaffold; baseline (speedup 1.0000x reference)
import jax
import jax.numpy as jnp
from jax.experimental import pallas as pl


def kernel(x, edge_index, W1, b1, W2, b2):
    raise NotImplementedError("write your pallas kernel here")


# speedup vs baseline: 31.8372x; 31.8372x over previous
import functools

import jax
import jax.numpy as jnp
from jax import lax
from jax.experimental import pallas as pl
from jax.experimental.pallas import tpu as pltpu
from jax.experimental.pallas import tpu_sc as plsc

N = 10000
D = 128
E = 320000
NC = 2
NS = 16
NW = NC * NS
LANES = 16
NPAD = 10240
NROW = NPAD // 128
EW = E // NW
CH = 80
NCH = EW // CH
ARS = NPAD // NS
SEG = NPAD // NS

_mesh = plsc.VectorSubcoreMesh(core_axis_name="c", subcore_axis_name="s")


def _zero_flat(ref, n):
    zero16 = jnp.zeros((LANES,), jnp.float32)

    @pl.loop(0, n // LANES)
    def _(i):
        ref[pl.ds(i * LANES, LANES)] = zero16


def _zero_rows(ref, nrows, width):
    zero16 = jnp.zeros((LANES,), jnp.float32)

    @pl.loop(0, nrows)
    def _(r):
        for k in range(width // LANES):
            ref[r, pl.ds(k * LANES, LANES)] = zero16


def _reduce_hist(hist_v, stage_sh, buf_v, out_v, out_hbm, c, s):
    pltpu.sync_copy(hist_v, stage_sh.at[pl.ds(s * NPAD, NPAD)])
    plsc.subcore_barrier()
    for r in range(NS):
        pltpu.sync_copy(stage_sh.at[pl.ds(r * NPAD + s * SEG, SEG)],
                        buf_v.at[pl.ds(r * SEG, SEG)])

    @pl.loop(0, SEG // LANES)
    def _(g):
        acc = buf_v[pl.ds(g * LANES, LANES)]
        for r in range(1, NS):
            acc = acc + buf_v[pl.ds(r * SEG + g * LANES, LANES)]
        out_v[pl.ds(g * LANES, LANES)] = acc

    pltpu.sync_copy(out_v, out_hbm.at[c, pl.ds(s * SEG, SEG)])


@functools.partial(
    pl.kernel,
    out_type=jax.ShapeDtypeStruct((NC, NPAD), jnp.float32),
    mesh=_mesh,
    compiler_params=pltpu.CompilerParams(needs_layout_passes=False),
    scratch_types=[
        pltpu.VMEM((NCH, CH), jnp.int32),
        pltpu.VMEM((NPAD,), jnp.float32),
        pltpu.VMEM((NS * SEG,), jnp.float32),
        pltpu.VMEM((SEG,), jnp.float32),
        pltpu.VMEM_SHARED((NS * NPAD,), jnp.float32),
    ],
)
def _deg_kernel(dst_hbm, out_hbm, dst_v, hist_v, buf_v, out_v, stage_sh):
    c = lax.axis_index("c")
    s = lax.axis_index("s")
    wid = c * NS + s
    pltpu.sync_copy(dst_hbm.at[wid], dst_v)
    _zero_flat(hist_v, NPAD)
    ones16 = jnp.ones((LANES,), jnp.float32)

    @pl.loop(0, NCH)
    def _(j):
        for k in range(CH // LANES):
            d16 = dst_v[j, pl.ds(k * LANES, LANES)]
            plsc.addupdate_scatter(hist_v, [d16], ones16)

    _reduce_hist(hist_v, stage_sh, buf_v, out_v, out_hbm, c, s)


@functools.partial(
    pl.kernel,
    out_type=jax.ShapeDtypeStruct((NC, NPAD, D), jnp.float32),
    mesh=_mesh,
    compiler_params=pltpu.CompilerParams(needs_layout_passes=False),
    scratch_types=[
        pltpu.VMEM((NCH, CH), jnp.int32),
        pltpu.VMEM((NCH, CH), jnp.int32),
        pltpu.VMEM((CH, D), jnp.float32),
        pltpu.VMEM_SHARED((NPAD, D), jnp.float32),
        pltpu.SemaphoreType.DMA,
    ],
)
def _agg_kernel(src_hbm, dst_hbm, t1_hbm, out_hbm, src_v, dst_v, rows_v, acc_sh, sem):
    c = lax.axis_index("c")
    s = lax.axis_index("s")
    wid = c * NS + s
    pltpu.sync_copy(src_hbm.at[wid], src_v)
    pltpu.sync_copy(dst_hbm.at[wid], dst_v)
    _zero_rows(rows_v, CH, D)
    for k in range(ARS // CH):
        pltpu.sync_copy(rows_v, acc_sh.at[pl.ds(s * ARS + k * CH, CH)])
    plsc.subcore_barrier()

    @pl.loop(0, NCH)
    def _(j):
        pltpu.async_copy(t1_hbm.at[src_v.at[j]], rows_v, sem).wait()
        pltpu.sync_copy(rows_v, acc_sh.at[dst_v.at[j]], add=True)

    plsc.subcore_barrier()
    for k in range(ARS // CH):
        pltpu.sync_copy(acc_sh.at[pl.ds(s * ARS + k * CH, CH)],
                        out_hbm.at[c, pl.ds(s * ARS + k * CH, CH)])


@functools.partial(
    pl.kernel,
    out_type=jax.ShapeDtypeStruct((NC, NPAD), jnp.float32),
    mesh=_mesh,
    compiler_params=pltpu.CompilerParams(needs_layout_passes=False),
    scratch_types=[
        pltpu.VMEM((NCH, CH), jnp.int32),
        pltpu.VMEM((NCH, CH), jnp.int32),
        pltpu.VMEM((N,), jnp.float32),
        pltpu.VMEM((NPAD,), jnp.float32),
        pltpu.VMEM((NS * SEG,), jnp.float32),
        pltpu.VMEM((SEG,), jnp.float32),
        pltpu.VMEM_SHARED((NS * NPAD,), jnp.float32),
    ],
)
def _seg_kernel(src_hbm, dst_hbm, t2_hbm, out_hbm, src_v, dst_v, t2_v, hist_v,
                buf_v, out_v, stage_sh):
    c = lax.axis_index("c")
    s = lax.axis_index("s")
    wid = c * NS + s
    pltpu.sync_copy(src_hbm.at[wid], src_v)
    pltpu.sync_copy(dst_hbm.at[wid], dst_v)
    pltpu.sync_copy(t2_hbm, t2_v)
    _zero_flat(hist_v, NPAD)

    @pl.loop(0, NCH)
    def _(j):
        for k in range(CH // LANES):
            s16 = src_v[j, pl.ds(k * LANES, LANES)]
            d16 = dst_v[j, pl.ds(k * LANES, LANES)]
            vals = plsc.load_gather(t2_v, [s16])
            plsc.addupdate_scatter(hist_v, [d16], vals)

    _reduce_hist(hist_v, stage_sh, buf_v, out_v, out_hbm, c, s)


def _mm1_body(x_ref, w1_ref, p0_ref, p1_ref, t1_ref, dinv_ref):
    deg = p0_ref[...] + p1_ref[...] + 1.0
    dinv = lax.rsqrt(jnp.maximum(deg, 1.0))
    dinv_ref[...] = dinv
    t1_ref[...] = jnp.dot(x_ref[...] * dinv, w1_ref[...],
                          precision=lax.Precision.HIGHEST,
                          preferred_element_type=jnp.float32)


_mm1 = pl.pallas_call(
    _mm1_body,
    out_shape=(jax.ShapeDtypeStruct((N, D), jnp.float32),
               jax.ShapeDtypeStruct((N, 1), jnp.float32)),
)


def _mm2_body(a0_ref, a1_ref, t1_ref, dinv_ref, b1_ref, w2_ref, t2_ref):
    agg = a0_ref[...] + a1_ref[...] + t1_ref[...]
    h = jnp.maximum(agg * dinv_ref[...] + b1_ref[...], 0.0)
    t2_ref[...] = jnp.dot(h, w2_ref[...],
                          precision=lax.Precision.HIGHEST,
                          preferred_element_type=jnp.float32) * dinv_ref[...]


_mm2 = pl.pallas_call(
    _mm2_body,
    out_shape=jax.ShapeDtypeStruct((N, 1), jnp.float32),
)


def _fin_body(e0_ref, e1_ref, t2_ref, dinv_ref, b2_ref, out_ref):
    out_ref[...] = ((e0_ref[...] + e1_ref[...] + t2_ref[...]) * dinv_ref[...]
                    + b2_ref[...])


_fin = pl.pallas_call(
    _fin_body,
    out_shape=jax.ShapeDtypeStruct((N, 1), jnp.float32),
)


def kernel(x, edge_index, W1, b1, W2, b2):
    e32 = edge_index.astype(jnp.int32).reshape(2, NW, NCH, CH)
    src_r, dst_r = e32[0], e32[1]
    degp = _deg_kernel(dst_r)
    p = degp.reshape(NC, NPAD, 1)[:, :N]
    t1, dinv = _mm1(x, W1, p[0], p[1])
    aggp = _agg_kernel(src_r, dst_r, t1)
    t2 = _mm2(aggp[0, :N], aggp[1, :N], t1, dinv, b1.reshape(1, D), W2)
    segp = _seg_kernel(src_r, dst_r, t2.reshape(N))
    e = segp.reshape(NC, NPAD, 1)[:, :N]
    return _fin(e[0], e[1], t2, dinv, b2.reshape(1, 1))


# speedup vs baseline: 42.6071x; 1.3383x over previous
import functools

import jax
import jax.numpy as jnp
from jax import lax
from jax.experimental import pallas as pl
from jax.experimental.pallas import tpu as pltpu
from jax.experimental.pallas import tpu_sc as plsc

N = 10000
D = 128
E = 320000
NC = 2
NS = 16
NW = NC * NS
LANES = 16
NPAD = 10240
NROW = NPAD // 128
EW = E // NW
CH = 80
NCH = EW // CH
VR = EW // LANES
ARS = NPAD // NS
SEG = NPAD // NS

_mesh = plsc.VectorSubcoreMesh(core_axis_name="c", subcore_axis_name="s")


def _zero_flat(ref, n):
    zero16 = jnp.zeros((LANES,), jnp.float32)

    @pl.loop(0, n // LANES)
    def _(i):
        ref[pl.ds(i * LANES, LANES)] = zero16


def _zero_rows(ref, nrows, width):
    zero16 = jnp.zeros((LANES,), jnp.float32)

    @pl.loop(0, nrows)
    def _(r):
        for k in range(width // LANES):
            ref[r, pl.ds(k * LANES, LANES)] = zero16


def _reduce_hist(hist_v, stage_sh, buf_v, out_v, out_hbm, c, s):
    pltpu.sync_copy(hist_v, stage_sh.at[pl.ds(s * NPAD, NPAD)])
    plsc.subcore_barrier()
    for r in range(NS):
        pltpu.sync_copy(stage_sh.at[pl.ds(r * NPAD + s * SEG, SEG)],
                        buf_v.at[pl.ds(r * SEG, SEG)])

    @pl.loop(0, SEG // LANES)
    def _(g):
        acc = buf_v[pl.ds(g * LANES, LANES)]
        for r in range(1, NS):
            acc = acc + buf_v[pl.ds(r * SEG + g * LANES, LANES)]
        out_v[pl.ds(g * LANES, LANES)] = acc

    pltpu.sync_copy(out_v, out_hbm.at[c, pl.ds(s * SEG, SEG)])


@functools.partial(
    pl.kernel,
    out_type=jax.ShapeDtypeStruct((NC, NPAD), jnp.float32),
    mesh=_mesh,
    compiler_params=pltpu.CompilerParams(needs_layout_passes=False),
    scratch_types=[
        pltpu.VMEM((EW,), jnp.int32),
        pltpu.VMEM((NPAD,), jnp.float32),
        pltpu.VMEM((NS * SEG,), jnp.float32),
        pltpu.VMEM((SEG,), jnp.float32),
        pltpu.VMEM_SHARED((NS * NPAD,), jnp.float32),
    ],
)
def _deg_kernel(dst_hbm, out_hbm, dst_v, hist_v, buf_v, out_v, stage_sh):
    c = lax.axis_index("c")
    s = lax.axis_index("s")
    wid = c * NS + s
    pltpu.sync_copy(dst_hbm.at[wid], dst_v)
    _zero_flat(hist_v, NPAD)
    ones16 = jnp.ones((LANES,), jnp.float32)

    @pl.loop(0, VR)
    def _(j):
        d16 = dst_v[pl.ds(j * LANES, LANES)]
        plsc.addupdate_scatter(hist_v, [d16], ones16)

    _reduce_hist(hist_v, stage_sh, buf_v, out_v, out_hbm, c, s)


@functools.partial(
    pl.kernel,
    out_type=jax.ShapeDtypeStruct((NC, NPAD, D), jnp.float32),
    mesh=_mesh,
    compiler_params=pltpu.CompilerParams(needs_layout_passes=False),
    scratch_types=[
        pltpu.VMEM((EW,), jnp.int32),
        pltpu.VMEM((NCH, CH), jnp.int32),
        pltpu.VMEM((CH, D), jnp.float32),
        pltpu.VMEM((CH, D), jnp.float32),
        pltpu.VMEM_SHARED((NPAD, D), jnp.float32),
        pltpu.SemaphoreType.DMA,
        pltpu.SemaphoreType.DMA,
    ],
)
def _agg_kernel(src_hbm, dst_hbm, t1_hbm, out_hbm, src_v, dst_v, buf0_v, buf1_v,
                acc_sh, sem0, sem1):
    c = lax.axis_index("c")
    s = lax.axis_index("s")
    wid = c * NS + s
    pltpu.sync_copy(src_hbm.at[wid], src_v)
    pltpu.sync_copy(dst_hbm.at[wid], dst_v)
    _zero_rows(buf0_v, CH, D)
    for k in range(ARS // CH):
        pltpu.sync_copy(buf0_v, acc_sh.at[pl.ds(s * ARS + k * CH, CH)])
    plsc.subcore_barrier()

    def _start(j, buf, sem):
        pltpu.async_copy(t1_hbm.at[src_v.at[pl.ds(j * CH, CH)]], buf, sem)

    def _wait(buf, sem):
        pltpu.make_async_copy(t1_hbm.at[pl.ds(0, CH)], buf, sem).wait()

    _start(0, buf0_v, sem0)

    @pl.loop(0, (NCH - 1) // 2)
    def _(jj):
        j0 = jj * 2
        _start(j0 + 1, buf1_v, sem1)
        _wait(buf0_v, sem0)
        pltpu.sync_copy(buf0_v, acc_sh.at[dst_v.at[j0]], add=True)
        _start(j0 + 2, buf0_v, sem0)
        _wait(buf1_v, sem1)
        pltpu.sync_copy(buf1_v, acc_sh.at[dst_v.at[j0 + 1]], add=True)

    _wait(buf0_v, sem0)
    pltpu.sync_copy(buf0_v, acc_sh.at[dst_v.at[NCH - 1]], add=True)
    plsc.subcore_barrier()
    for k in range(ARS // CH):
        pltpu.sync_copy(acc_sh.at[pl.ds(s * ARS + k * CH, CH)],
                        out_hbm.at[c, pl.ds(s * ARS + k * CH, CH)])


@functools.partial(
    pl.kernel,
    out_type=jax.ShapeDtypeStruct((NC, NPAD), jnp.float32),
    mesh=_mesh,
    compiler_params=pltpu.CompilerParams(needs_layout_passes=False),
    scratch_types=[
        pltpu.VMEM((EW,), jnp.int32),
        pltpu.VMEM((EW,), jnp.int32),
        pltpu.VMEM((N,), jnp.float32),
        pltpu.VMEM((NPAD,), jnp.float32),
        pltpu.VMEM((NS * SEG,), jnp.float32),
        pltpu.VMEM((SEG,), jnp.float32),
        pltpu.VMEM_SHARED((NS * NPAD,), jnp.float32),
    ],
)
def _seg_kernel(src_hbm, dst_hbm, t2_hbm, out_hbm, src_v, dst_v, t2_v, hist_v,
                buf_v, out_v, stage_sh):
    c = lax.axis_index("c")
    s = lax.axis_index("s")
    wid = c * NS + s
    pltpu.sync_copy(src_hbm.at[wid], src_v)
    pltpu.sync_copy(dst_hbm.at[wid], dst_v)
    pltpu.sync_copy(t2_hbm, t2_v)
    _zero_flat(hist_v, NPAD)

    @pl.loop(0, VR)
    def _(j):
        s16 = src_v[pl.ds(j * LANES, LANES)]
        d16 = dst_v[pl.ds(j * LANES, LANES)]
        vals = plsc.load_gather(t2_v, [s16])
        plsc.addupdate_scatter(hist_v, [d16], vals)

    _reduce_hist(hist_v, stage_sh, buf_v, out_v, out_hbm, c, s)


def _mm1_body(x_ref, w1_ref, p0_ref, p1_ref, t1_ref, dinv_ref):
    deg = p0_ref[...] + p1_ref[...] + 1.0
    dinv = lax.rsqrt(jnp.maximum(deg, 1.0))
    dinv_ref[...] = dinv
    t1_ref[...] = jnp.dot(x_ref[...] * dinv, w1_ref[...],
                          precision=lax.Precision.HIGHEST,
                          preferred_element_type=jnp.float32)


_mm1 = pl.pallas_call(
    _mm1_body,
    out_shape=(jax.ShapeDtypeStruct((N, D), jnp.float32),
               jax.ShapeDtypeStruct((N, 1), jnp.float32)),
)


def _mm2_body(a0_ref, a1_ref, t1_ref, dinv_ref, b1_ref, w2_ref, t2_ref):
    agg = a0_ref[...] + a1_ref[...] + t1_ref[...]
    h = jnp.maximum(agg * dinv_ref[...] + b1_ref[...], 0.0)
    t2_ref[...] = jnp.dot(h, w2_ref[...],
                          precision=lax.Precision.HIGHEST,
                          preferred_element_type=jnp.float32) * dinv_ref[...]


_mm2 = pl.pallas_call(
    _mm2_body,
    out_shape=jax.ShapeDtypeStruct((N, 1), jnp.float32),
)


def _fin_body(e0_ref, e1_ref, t2_ref, dinv_ref, b2_ref, out_ref):
    out_ref[...] = ((e0_ref[...] + e1_ref[...] + t2_ref[...]) * dinv_ref[...]
                    + b2_ref[...])


_fin = pl.pallas_call(
    _fin_body,
    out_shape=jax.ShapeDtypeStruct((N, 1), jnp.float32),
)


def kernel(x, edge_index, W1, b1, W2, b2):
    e32 = edge_index.astype(jnp.int32)
    e_agg = e32.reshape(2, NW, NCH, CH)
    e_vr = e32.reshape(2, NW, EW)
    degp = _deg_kernel(e_vr[1])
    p = degp.reshape(NC, NPAD, 1)[:, :N]
    t1, dinv = _mm1(x, W1, p[0], p[1])
    aggp = _agg_kernel(e_vr[0], e_agg[1], t1)
    t2 = _mm2(aggp[0, :N], aggp[1, :N], t1, dinv, b1.reshape(1, D), W2)
    segp = _seg_kernel(e_vr[0], e_vr[1], t2.reshape(N))
    e = segp.reshape(NC, NPAD, 1)[:, :N]
    return _fin(e[0], e[1], t2, dinv, b2.reshape(1, 1))
